# baseline (speedup 1.0000x reference)
import functools

import jax
import jax.numpy as jnp
from jax import lax
from jax.experimental import pallas as pl
from jax.experimental.pallas import tpu as pltpu
from jax.experimental.pallas import tpu_sc as plsc


_RBLK = 128


def _argmax_body(p_ref, out_ref):
    x = p_ref[...]
    bm = jnp.max(x, axis=1, keepdims=True)
    col = jax.lax.broadcasted_iota(jnp.int32, x.shape, 1)
    big = jnp.int32(2**31 - 1)
    out_ref[...] = jnp.min(jnp.where(x == bm, col, big), axis=1, keepdims=True)


def _rowwise_argmax(parameter):
    n_rows, n_cols = parameter.shape
    grid = (n_rows // _RBLK,)
    ind2d = pl.pallas_call(
        _argmax_body,
        grid=grid,
        in_specs=[pl.BlockSpec((_RBLK, n_cols), lambda i: (i, 0))],
        out_specs=pl.BlockSpec((_RBLK, 1), lambda i: (i, 0)),
        out_shape=jax.ShapeDtypeStruct((n_rows, 1), jnp.int32),
    )(parameter)
    return ind2d.reshape(n_rows)


def _make_sc_gather(V, D, B):
    info = plsc.get_sparse_core_info()
    NC, NS = info.num_cores, info.num_subcores
    NW = NC * NS
    assert B % (8 * NW) == 0
    b_per_w = B // NW
    mesh = plsc.VectorSubcoreMesh(core_axis_name="c", subcore_axis_name="s")

    @functools.partial(
        pl.kernel,
        mesh=mesh,
        out_type=jax.ShapeDtypeStruct((B, D), jnp.float32),
        scratch_types=[
            pltpu.VMEM((b_per_w,), jnp.int32),
            pltpu.VMEM((b_per_w, D), jnp.float32),
            pltpu.SemaphoreType.DMA,
        ],
    )
    def gather_k(table_hbm, idx_hbm, out_hbm, idx_v, rows_v, sem):
        wid = lax.axis_index("s") * NC + lax.axis_index("c")
        base = wid * b_per_w
        pltpu.sync_copy(idx_hbm.at[pl.ds(base, b_per_w)], idx_v)
        pltpu.async_copy(table_hbm.at[idx_v], rows_v, sem).wait()
        pltpu.sync_copy(rows_v, out_hbm.at[pl.ds(base, b_per_w)])

    return gather_k


def kernel(inputs, parameter):
    ind = _rowwise_argmax(parameter)
    table = inputs.T
    V, D = table.shape
    B = ind.shape[0]
    rows = _make_sc_gather(V, D, B)(table, ind)
    return rows.T


# speedup vs baseline: 1.0088x; 1.0088x over previous
import functools

import jax
import jax.numpy as jnp
from jax import lax
from jax.experimental import pallas as pl
from jax.experimental.pallas import tpu as pltpu
from jax.experimental.pallas import tpu_sc as plsc


_RBLK = 128


def _argmax_body(p_ref, in_ref, out_ref, tab_ref):
    x = p_ref[...]
    bm = jnp.max(x, axis=1, keepdims=True)
    col = jax.lax.broadcasted_iota(jnp.int32, x.shape, 1)
    big = jnp.int32(2**31 - 1)
    out_ref[...] = jnp.min(jnp.where(x == bm, col, big), axis=1, keepdims=True)
    tab_ref[...] = in_ref[...].T


def _rowwise_argmax(parameter, inputs):
    n_rows, n_cols = parameter.shape
    n_b, n_in = inputs.shape
    grid = (n_rows // _RBLK,)
    tcol = n_in // grid[0]
    ind2d, table = pl.pallas_call(
        _argmax_body,
        grid=grid,
        in_specs=[
            pl.BlockSpec((_RBLK, n_cols), lambda i: (i, 0)),
            pl.BlockSpec((n_b, tcol), lambda i: (0, i)),
        ],
        out_specs=[
            pl.BlockSpec((_RBLK, 1), lambda i: (i, 0)),
            pl.BlockSpec((tcol, n_b), lambda i: (i, 0)),
        ],
        out_shape=[
            jax.ShapeDtypeStruct((n_rows, 1), jnp.int32),
            jax.ShapeDtypeStruct((n_in, n_b), jnp.float32),
        ],
    )(parameter, inputs)
    return ind2d.reshape(n_rows), table


def _make_sc_gather(V, D, B):
    info = plsc.get_sparse_core_info()
    NC, NS = info.num_cores, info.num_subcores
    NW = NC * NS
    assert B % (8 * NW) == 0
    b_per_w = B // NW
    mesh = plsc.VectorSubcoreMesh(core_axis_name="c", subcore_axis_name="s")

    @functools.partial(
        pl.kernel,
        mesh=mesh,
        out_type=jax.ShapeDtypeStruct((B, D), jnp.float32),
        scratch_types=[
            pltpu.VMEM((b_per_w,), jnp.int32),
            pltpu.VMEM((b_per_w, D), jnp.float32),
            pltpu.SemaphoreType.DMA,
        ],
    )
    def gather_k(table_hbm, idx_hbm, out_hbm, idx_v, rows_v, sem):
        wid = lax.axis_index("s") * NC + lax.axis_index("c")
        base = wid * b_per_w
        pltpu.sync_copy(idx_hbm.at[pl.ds(base, b_per_w)], idx_v)
        pltpu.async_copy(table_hbm.at[idx_v], rows_v, sem).wait()
        pltpu.sync_copy(rows_v, out_hbm.at[pl.ds(base, b_per_w)])

    return gather_k


def kernel(inputs, parameter):
    ind, table = _rowwise_argmax(parameter, inputs)
    V, D = table.shape
    B = ind.shape[0]
    rows = _make_sc_gather(V, D, B)(table, ind)
    return rows.T
